# baseline (speedup 1.0000x reference)
import jax
import jax.numpy as jnp
from jax import lax
from jax.experimental import pallas as pl
from jax.experimental.pallas import tpu as pltpu
from jax.experimental.pallas import tpu_sc as plsc


def kernel(features, labels, features_memory, labels_memory):
    batch = features.shape[0]
    dim = features_memory.shape[1]

    fmT = jnp.transpose(features_memory)
    lmT = jnp.transpose(labels_memory)

    mesh = plsc.VectorSubcoreMesh(core_axis_name="c", subcore_axis_name="s")
    num_workers = mesh.num_cores * mesh.num_subcores
    cols = batch // num_workers

    @pl.kernel(
        out_type=(
            jax.ShapeDtypeStruct((dim, batch), features_memory.dtype),
            jax.ShapeDtypeStruct((1, batch), labels_memory.dtype),
        ),
        mesh=mesh,
        scratch_types=[
            pltpu.VMEM((dim, cols), features_memory.dtype),
            pltpu.VMEM((1, cols), labels_memory.dtype),
            pltpu.SemaphoreType.DMA,
            pltpu.SemaphoreType.DMA,
        ],
    )
    def gather_occupied(fm_hbm, lm_hbm, fo_hbm, lo_hbm, fbuf, lbuf, sem_f, sem_l):
        c = lax.axis_index("c")
        s = lax.axis_index("s")
        wid = c * mesh.num_subcores + s
        start = wid * cols
        gf = pltpu.async_copy(fm_hbm.at[:, pl.ds(start, cols)], fbuf, sem_f)
        gl = pltpu.async_copy(lm_hbm.at[:, pl.ds(start, cols)], lbuf, sem_l)
        gf.wait()
        gl.wait()
        of = pltpu.async_copy(fbuf, fo_hbm.at[:, pl.ds(start, cols)], sem_f)
        ol = pltpu.async_copy(lbuf, lo_hbm.at[:, pl.ds(start, cols)], sem_l)
        of.wait()
        ol.wait()

    foT, loT = gather_occupied(fmT, lmT)
    return jnp.transpose(foT), jnp.transpose(loT)


# speedup vs baseline: 1.0126x; 1.0126x over previous
import jax
import jax.numpy as jnp
from jax import lax
from jax.experimental import pallas as pl
from jax.experimental.pallas import tpu as pltpu
from jax.experimental.pallas import tpu_sc as plsc


def _feat_body(fmT_ref, foT_ref):
    foT_ref[...] = fmT_ref[...]


def kernel(features, labels, features_memory, labels_memory):
    batch = features.shape[0]
    dim = features_memory.shape[1]

    fmT = jnp.transpose(features_memory)
    lmT = jnp.transpose(labels_memory)

    blk = 2048
    foT = pl.pallas_call(
        _feat_body,
        grid=(batch // blk,),
        out_shape=jax.ShapeDtypeStruct((dim, batch), features_memory.dtype),
        in_specs=[pl.BlockSpec((dim, blk), lambda i: (0, i))],
        out_specs=pl.BlockSpec((dim, blk), lambda i: (0, i)),
    )(fmT)

    mesh = plsc.VectorSubcoreMesh(core_axis_name="c", subcore_axis_name="s")
    num_workers = mesh.num_cores * mesh.num_subcores
    cols = batch // num_workers

    @pl.kernel(
        out_type=jax.ShapeDtypeStruct((1, batch), labels_memory.dtype),
        mesh=mesh,
        scratch_types=[
            pltpu.VMEM((1, cols), labels_memory.dtype),
            pltpu.SemaphoreType.DMA,
        ],
    )
    def labels_readback(lm_hbm, lo_hbm, lbuf, sem):
        c = lax.axis_index("c")
        s = lax.axis_index("s")
        wid = c * mesh.num_subcores + s
        start = wid * cols
        pltpu.async_copy(lm_hbm.at[:, pl.ds(start, cols)], lbuf, sem).wait()
        pltpu.async_copy(lbuf, lo_hbm.at[:, pl.ds(start, cols)], sem).wait()

    loT = labels_readback(lmT)
    return jnp.transpose(foT), jnp.transpose(loT)
